# Initial kernel scaffold; baseline (speedup 1.0000x reference)
import jax
import jax.numpy as jnp
from jax.experimental import pallas as pl


def kernel(x, W, b):
    raise NotImplementedError("write your pallas kernel here")


# speedup vs baseline: 44.7002x; 44.7002x over previous
import functools

import jax
import jax.numpy as jnp
from jax.experimental import pallas as pl
from jax.experimental.pallas import tpu as pltpu


def _stg_kernel(x_ref, w_ref, b_ref, out_ref, scores_ref, acc_ref, *,
                bk: int, topk: int):
    k = pl.program_id(1)
    nk = pl.num_programs(1)

    @pl.when(k == 0)
    def _init():
        acc_ref[...] = jnp.zeros_like(acc_ref)

    x_blk = x_ref[:, pl.ds(k * bk, bk)].astype(jnp.bfloat16)
    acc_ref[...] += jax.lax.dot_general(
        x_blk, w_ref[...].astype(jnp.bfloat16),
        dimension_numbers=(((1,), (1,)), ((), ())),
        preferred_element_type=jnp.float32)

    @pl.when(k == nk - 1)
    def _finish():
        scores = jax.nn.sigmoid(acc_ref[...] + b_ref[...])
        scores_ref[...] = scores
        keys = jax.lax.bitcast_convert_type(scores, jnp.int32)
        topk_f = jnp.float32(topk)

        def bit_step(i, mag):
            cand = mag + (jnp.int32(1) << (29 - i))
            cnt = jnp.sum((keys >= cand).astype(jnp.float32), axis=1,
                          keepdims=True)
            return jnp.where(cnt >= topk_f, cand, mag)

        thresh = jax.lax.fori_loop(
            0, 30, bit_step, jnp.zeros((keys.shape[0], 1), jnp.int32))
        out_ref[...] = jnp.where(keys >= thresh, x_ref[...], 0.0)


def kernel(x, W, b):
    m, kdim = x.shape
    n = W.shape[0]
    topk = max(1, int(0.3 * n))
    bm = min(256, m)
    bk = min(512, kdim)
    grid = (m // bm, kdim // bk)

    masked, scores = pl.pallas_call(
        functools.partial(_stg_kernel, bk=bk, topk=topk),
        grid=grid,
        in_specs=[
            pl.BlockSpec((bm, kdim), lambda i, k: (i, 0)),
            pl.BlockSpec((n, bk), lambda i, k: (0, k)),
            pl.BlockSpec((1, n), lambda i, k: (0, 0)),
        ],
        out_specs=[
            pl.BlockSpec((bm, n), lambda i, k: (i, 0)),
            pl.BlockSpec((bm, n), lambda i, k: (i, 0)),
        ],
        out_shape=[
            jax.ShapeDtypeStruct((m, n), jnp.float32),
            jax.ShapeDtypeStruct((m, n), jnp.float32),
        ],
        scratch_shapes=[pltpu.VMEM((bm, n), jnp.float32)],
        compiler_params=pltpu.CompilerParams(
            dimension_semantics=("parallel", "arbitrary")),
    )(x, W, b.reshape(1, n))
    return (masked, scores)


# speedup vs baseline: 49.1097x; 1.0986x over previous
import functools

import jax
import jax.numpy as jnp
from jax.experimental import pallas as pl
from jax.experimental.pallas import tpu as pltpu


def _stg_kernel(x_ref, w_ref, b_ref, out_ref, scores_ref, *,
                bk: int, topk: int):
    k = pl.program_id(1)
    nk = pl.num_programs(1)

    x_blk = x_ref[:, pl.ds(k * bk, bk)].astype(jnp.bfloat16)
    prod = jax.lax.dot_general(
        x_blk, w_ref[...],
        dimension_numbers=(((1,), (1,)), ((), ())),
        preferred_element_type=jnp.float32)

    @pl.when(k == 0)
    def _init():
        scores_ref[...] = prod

    @pl.when(k > 0)
    def _accum():
        scores_ref[...] += prod

    @pl.when(k == nk - 1)
    def _finish():
        scores = jax.nn.sigmoid(scores_ref[...] + b_ref[...])
        scores_ref[...] = scores
        keys = jax.lax.bitcast_convert_type(scores, jnp.int32)
        topk_f = jnp.float32(topk)

        def bit_step(i, mag):
            cand = mag + (jnp.int32(1) << (29 - i))
            cnt = jnp.sum((keys >= cand).astype(jnp.float32), axis=1,
                          keepdims=True)
            return jnp.where(cnt >= topk_f, cand, mag)

        thresh = jax.lax.fori_loop(
            0, 30, bit_step, jnp.zeros((keys.shape[0], 1), jnp.int32))
        out_ref[...] = jnp.where(keys >= thresh, x_ref[...], 0.0)


def kernel(x, W, b):
    m, kdim = x.shape
    n = W.shape[0]
    topk = max(1, int(0.3 * n))
    bm = min(256, m)
    bk = min(512, kdim)
    grid = (m // bm, kdim // bk)

    masked, scores = pl.pallas_call(
        functools.partial(_stg_kernel, bk=bk, topk=topk),
        grid=grid,
        in_specs=[
            pl.BlockSpec((bm, kdim), lambda i, k: (i, 0)),
            pl.BlockSpec((n, bk), lambda i, k: (0, k)),
            pl.BlockSpec((1, n), lambda i, k: (0, 0)),
        ],
        out_specs=[
            pl.BlockSpec((bm, n), lambda i, k: (i, 0)),
            pl.BlockSpec((bm, n), lambda i, k: (i, 0)),
        ],
        out_shape=[
            jax.ShapeDtypeStruct((m, n), jnp.float32),
            jax.ShapeDtypeStruct((m, n), jnp.float32),
        ],
        compiler_params=pltpu.CompilerParams(
            dimension_semantics=("parallel", "arbitrary")),
    )(x, W.astype(jnp.bfloat16), b.reshape(1, n))
    return (masked, scores)
